# baseline (speedup 1.0000x reference)
import functools

import jax
import jax.numpy as jnp
from jax import lax
from jax.experimental import pallas as pl
from jax.experimental.pallas import tpu as pltpu
from jax.experimental.pallas import tpu_sc as plsc

_NC = 2
_NS = 16
_LANES = 16


def _keys_body(cls_ref, den_ref, anc_ref, keys_ref):
    cls = cls_ref[0]
    den = den_ref[0]
    anc = anc_ref[0]

    eps = jnp.float32(1e-12)
    ssq = jnp.sum(cls * cls, axis=0, keepdims=True)
    inv_n = 1.0 / jnp.maximum(jnp.sqrt(ssq), eps)
    asq = jnp.sum(anc * anc, axis=1, keepdims=True)
    na = anc / jnp.maximum(jnp.sqrt(asq), eps)
    na_col = na.reshape(anc.shape[1], 1)

    sim = den * ((cls * inv_n) * na_col)

    bits = pltpu.bitcast(sim, jnp.uint32)
    keys_ref[0] = jnp.where(
        (bits >> 31) == jnp.uint32(0),
        bits | jnp.uint32(0x80000000),
        ~bits,
    )


def _sc_select_body(K, R, keys_hbm, cls_hbm, size_hbm, out_hbm,
                    keybuf, clsbuf, outbuf, sizebuf):
    wid = lax.axis_index("s") * _NC + lax.axis_index("c")
    base_row = wid * R
    NV8 = K // (8 * _LANES)

    pltpu.sync_copy(size_hbm, sizebuf)
    size_i = sizebuf[...]
    size_f = size_i.astype(jnp.float32)
    lane = lax.iota(jnp.int32, _LANES)

    def splat(x):
        return jnp.broadcast_to(x, (_LANES,))

    gdn = lax.GatherDimensionNumbers(
        offset_dims=(), collapsed_slice_dims=(0,), start_index_map=(0,))

    def lane_sum(v):
        for sh in (8, 4, 2, 1):
            perm = lax.gather(v, (lane ^ sh)[:, None], gdn, (1,),
                              mode=lax.GatherScatterMode.PROMISE_IN_BOUNDS)
            v = v + perm
        return v

    def row_body(r, resvec):
        row = base_row + r
        pltpu.sync_copy(keys_hbm.at[row], keybuf)
        pltpu.sync_copy(cls_hbm.at[row], clsbuf)

        def bit_body(i, t):
            bitv = jnp.full((_LANES,), 0x80000000, jnp.uint32) \
                >> splat(i.astype(jnp.uint32))
            cand = t | bitv

            def cnt_body(j, acc):
                o = j * (8 * _LANES)
                for u in range(8):
                    kv = keybuf[pl.ds(o + u * _LANES, _LANES)]
                    acc = acc + jnp.where(kv >= cand, 1, 0).astype(jnp.int32)
                return acc

            acc = lax.fori_loop(0, NV8, cnt_body,
                                jnp.zeros((_LANES,), jnp.int32))
            total = lane_sum(acc)
            return jnp.where(total >= size_i, cand, t)

        t = lax.fori_loop(0, 32, bit_body, jnp.zeros((_LANES,), jnp.uint32))

        def sum_body(j, carry):
            sge, sgt, cge, cgt = carry
            o = j * (8 * _LANES)
            for u in range(8):
                kv = keybuf[pl.ds(o + u * _LANES, _LANES)]
                cv = clsbuf[pl.ds(o + u * _LANES, _LANES)]
                ge = kv >= t
                gt = kv > t
                one = jnp.ones((_LANES,), jnp.float32)
                zero = jnp.zeros((_LANES,), jnp.float32)
                sge = sge + jnp.where(ge, cv, zero)
                sgt = sgt + jnp.where(gt, cv, zero)
                cge = cge + jnp.where(ge, one, zero)
                cgt = cgt + jnp.where(gt, one, zero)
            return sge, sgt, cge, cgt

        z = jnp.zeros((_LANES,), jnp.float32)
        sge, sgt, cge, cgt = lax.fori_loop(0, NV8, sum_body, (z, z, z, z))
        sge_t = lane_sum(sge)
        sgt_t = lane_sum(sgt)
        cge_t = lane_sum(cge)
        cgt_t = lane_sum(cgt)
        eq = jnp.maximum(cge_t - cgt_t, 1.0)
        rowsum = sgt_t + (sge_t - sgt_t) * ((size_f - cgt_t) / eq)

        resvec = jnp.where(lane == splat(r & 15), rowsum, resvec)

        @pl.when((r & 15) == 15)
        def _():
            outbuf[pl.ds(pl.multiple_of(r - 15, 16), 16)] = resvec

        return resvec

    lax.fori_loop(0, R, row_body, jnp.zeros((_LANES,), jnp.float32))
    pltpu.sync_copy(outbuf, out_hbm.at[pl.ds(base_row, R)])


def _tc_select_body(size_ref, cls_ref, den_ref, anc_ref, out_ref):
    cls = cls_ref[0]
    den = den_ref[0]
    anc = anc_ref[0]
    size_f = size_ref[0]

    eps = jnp.float32(1e-12)
    ssq = jnp.sum(cls * cls, axis=0, keepdims=True)
    inv_n = 1.0 / jnp.maximum(jnp.sqrt(ssq), eps)
    asq = jnp.sum(anc * anc, axis=1, keepdims=True)
    na = anc / jnp.maximum(jnp.sqrt(asq), eps)
    na_col = na.reshape(anc.shape[1], 1)

    sim = den * ((cls * inv_n) * na_col)

    bits = pltpu.bitcast(sim, jnp.uint32)
    ku = jnp.where(
        (bits >> 31) == jnp.uint32(0),
        bits | jnp.uint32(0x80000000),
        ~bits,
    )

    C = cls.shape[0]
    t = jnp.zeros((C, 1), jnp.uint32)
    for i in range(32):
        cand = t | jnp.uint32(1 << (31 - i))
        cnt = jnp.sum(jnp.where(ku >= cand, 1.0, 0.0), axis=1, keepdims=True)
        t = jnp.where(cnt >= size_f, cand, t)

    m_ge = ku >= t
    m_gt = ku > t
    sum_ge = jnp.sum(jnp.where(m_ge, cls, 0.0), axis=1, keepdims=True)
    sum_gt = jnp.sum(jnp.where(m_gt, cls, 0.0), axis=1, keepdims=True)
    cnt_ge = jnp.sum(jnp.where(m_ge, 1.0, 0.0), axis=1, keepdims=True)
    cnt_gt = jnp.sum(jnp.where(m_gt, 1.0, 0.0), axis=1, keepdims=True)
    eq_cnt = jnp.maximum(cnt_ge - cnt_gt, 1.0)
    need = size_f - cnt_gt
    out_ref[0] = sum_gt + (sum_ge - sum_gt) * (need / eq_cnt)


def _emb_body(size_ref, denom_ref, asum_ref, anc_ref, w1_ref, b1_ref,
              w2_ref, b2_ref, out_ref):
    size_f = size_ref[0, 0]
    denom = denom_ref[0, 0]
    asum = asum_ref[...]
    anc = anc_ref[...]
    C = asum.shape[1]
    dn = functools.partial(
        jax.lax.dot_general,
        dimension_numbers=(((1,), (1,)), ((), ())),
        preferred_element_type=jnp.float32,
        precision=jax.lax.Precision.HIGHEST,
    )
    emb1 = (dn(asum, w1_ref[...]) + size_f * b1_ref[...]) / denom
    w2a = w2_ref[:, :C]
    w2b = w2_ref[:, C:]
    out_ref[...] = dn(anc, w2a) + dn(emb1, w2b) + b2_ref[...]


def kernel(clsfea, denfea, anchor_cls, bs_mean, b, W_emb, b_emb, W_emb2,
           b_emb2, interpret=False):
    Bc, C, H, Wd = clsfea.shape
    K = H * Wd
    a2 = 384.0 * 576.0 / H / Wd
    prod = bs_mean[0, 0] * bs_mean[0, 1]
    size = jnp.floor_divide(prod.astype(jnp.float32),
                            jnp.float32(a2)).astype(jnp.int32)
    size = jnp.maximum(size, 3)
    size_f = size.astype(jnp.float32).reshape(1, 1)
    denom = (size.astype(jnp.float32)
             * (jnp.asarray(b, jnp.float32) / Bc)).reshape(1, 1)

    cls3 = clsfea.reshape(Bc, C, K)
    den3 = denfea.reshape(Bc, C, K)
    anc3 = anchor_cls.reshape(Bc, 1, C)

    B_SC = 4
    B_TC = Bc - B_SC

    keys = pl.pallas_call(
        _keys_body,
        grid=(B_SC,),
        in_specs=[
            pl.BlockSpec((1, C, K), lambda i: (i, 0, 0)),
            pl.BlockSpec((1, C, K), lambda i: (i, 0, 0)),
            pl.BlockSpec((1, 1, C), lambda i: (i, 0, 0)),
        ],
        out_specs=pl.BlockSpec((1, C, K), lambda i: (i, 0, 0)),
        out_shape=jax.ShapeDtypeStruct((B_SC, C, K), jnp.uint32),
        interpret=interpret,
    )(cls3, den3, anc3)

    nrows = B_SC * C
    R = nrows // (_NC * _NS)
    size_vec = jnp.full((_LANES,), size, jnp.int32)
    mesh = plsc.VectorSubcoreMesh(core_axis_name="c", subcore_axis_name="s")
    sc_select = pl.kernel(
        functools.partial(_sc_select_body, K, R),
        mesh=mesh,
        out_type=jax.ShapeDtypeStruct((nrows,), jnp.float32),
        scratch_types=[
            pltpu.VMEM((K,), jnp.uint32),
            pltpu.VMEM((K,), jnp.float32),
            pltpu.VMEM((R,), jnp.float32),
            pltpu.VMEM((_LANES,), jnp.int32),
        ],
    )
    asum_sc = sc_select(keys.reshape(nrows, K),
                        clsfea.reshape(Bc * C, K), size_vec)

    asum_tc = pl.pallas_call(
        _tc_select_body,
        grid=(B_TC,),
        in_specs=[
            pl.BlockSpec((1, 1), lambda i: (0, 0)),
            pl.BlockSpec((1, C, K), lambda i: (i + B_SC, 0, 0)),
            pl.BlockSpec((1, C, K), lambda i: (i + B_SC, 0, 0)),
            pl.BlockSpec((1, 1, C), lambda i: (i + B_SC, 0, 0)),
        ],
        out_specs=pl.BlockSpec((1, C, 1), lambda i: (i, 0, 0)),
        out_shape=jax.ShapeDtypeStruct((B_TC, C, 1), jnp.float32),
        interpret=interpret,
    )(size_f, cls3, den3, anc3)

    asum = jnp.concatenate(
        [asum_sc.reshape(B_SC, C), asum_tc.reshape(B_TC, C)], axis=0)

    out = pl.pallas_call(
        _emb_body,
        out_shape=jax.ShapeDtypeStruct((Bc, C), jnp.float32),
        interpret=interpret,
    )(size_f, denom, asum, anchor_cls.reshape(Bc, C),
      W_emb, b_emb.reshape(1, C), W_emb2, b_emb2.reshape(1, C))

    return out.reshape(Bc, C, 1, 1)


# speedup vs baseline: 1.1809x; 1.1809x over previous
import functools

import jax
import jax.numpy as jnp
from jax import lax
from jax.experimental import pallas as pl
from jax.experimental.pallas import tpu as pltpu
from jax.experimental.pallas import tpu_sc as plsc

_NC = 2
_NS = 16
_LANES = 16


def _keys_body(cls_ref, den_ref, anc_ref, keys_ref):
    cls = cls_ref[0]
    den = den_ref[0]
    anc = anc_ref[0]

    eps = jnp.float32(1e-12)
    ssq = jnp.sum(cls * cls, axis=0, keepdims=True)
    inv_n = 1.0 / jnp.maximum(jnp.sqrt(ssq), eps)
    asq = jnp.sum(anc * anc, axis=1, keepdims=True)
    na = anc / jnp.maximum(jnp.sqrt(asq), eps)
    na_col = na.reshape(anc.shape[1], 1)

    sim = den * ((cls * inv_n) * na_col)

    bits = pltpu.bitcast(sim, jnp.uint32)
    keys_ref[0] = jnp.where(
        (bits >> 31) == jnp.uint32(0),
        bits | jnp.uint32(0x80000000),
        ~bits,
    )


def _sc_select_body(K, R, keys_hbm, cls_hbm, size_hbm, out_hbm,
                    keybuf, clsbuf, outbuf, sizebuf):
    wid = lax.axis_index("s") * _NC + lax.axis_index("c")
    base_row = wid * R
    NV8 = K // (8 * _LANES)

    pltpu.sync_copy(size_hbm, sizebuf)
    size_i = sizebuf[...]
    size_f = size_i.astype(jnp.float32)
    lane = lax.iota(jnp.int32, _LANES)

    def splat(x):
        return jnp.broadcast_to(x, (_LANES,))

    gdn = lax.GatherDimensionNumbers(
        offset_dims=(), collapsed_slice_dims=(0,), start_index_map=(0,))

    def lane_sum(v):
        for sh in (8, 4, 2, 1):
            perm = lax.gather(v, (lane ^ sh)[:, None], gdn, (1,),
                              mode=lax.GatherScatterMode.PROMISE_IN_BOUNDS)
            v = v + perm
        return v

    def row_body(r, resvec):
        row = base_row + r
        pltpu.sync_copy(keys_hbm.at[row], keybuf)
        pltpu.sync_copy(cls_hbm.at[row], clsbuf)

        def bit_body(i, t):
            bitv = jnp.full((_LANES,), 0x80000000, jnp.uint32) \
                >> splat(i.astype(jnp.uint32))
            cand = t | bitv

            def cnt_body(j, acc):
                o = j * (8 * _LANES)
                for u in range(8):
                    kv = keybuf[pl.ds(o + u * _LANES, _LANES)]
                    acc = acc + jnp.where(kv >= cand, 1, 0).astype(jnp.int32)
                return acc

            acc = lax.fori_loop(0, NV8, cnt_body,
                                jnp.zeros((_LANES,), jnp.int32))
            total = lane_sum(acc)
            return jnp.where(total >= size_i, cand, t)

        t = lax.fori_loop(0, 24, bit_body, jnp.zeros((_LANES,), jnp.uint32))

        def sum_body(j, carry):
            sge, sgt, cge, cgt = carry
            o = j * (8 * _LANES)
            for u in range(8):
                kv = keybuf[pl.ds(o + u * _LANES, _LANES)]
                cv = clsbuf[pl.ds(o + u * _LANES, _LANES)]
                ge = kv >= t
                gt = kv > t
                one = jnp.ones((_LANES,), jnp.float32)
                zero = jnp.zeros((_LANES,), jnp.float32)
                sge = sge + jnp.where(ge, cv, zero)
                sgt = sgt + jnp.where(gt, cv, zero)
                cge = cge + jnp.where(ge, one, zero)
                cgt = cgt + jnp.where(gt, one, zero)
            return sge, sgt, cge, cgt

        z = jnp.zeros((_LANES,), jnp.float32)
        sge, sgt, cge, cgt = lax.fori_loop(0, NV8, sum_body, (z, z, z, z))
        sge_t = lane_sum(sge)
        sgt_t = lane_sum(sgt)
        cge_t = lane_sum(cge)
        cgt_t = lane_sum(cgt)
        eq = jnp.maximum(cge_t - cgt_t, 1.0)
        rowsum = sgt_t + (sge_t - sgt_t) * ((size_f - cgt_t) / eq)

        resvec = jnp.where(lane == splat(r & 15), rowsum, resvec)

        @pl.when((r & 15) == 15)
        def _():
            outbuf[pl.ds(pl.multiple_of(r - 15, 16), 16)] = resvec

        return resvec

    lax.fori_loop(0, R, row_body, jnp.zeros((_LANES,), jnp.float32))
    pltpu.sync_copy(outbuf, out_hbm.at[pl.ds(base_row, R)])


def _tc_select_body(size_ref, cls_ref, den_ref, anc_ref, out_ref):
    cls = cls_ref[0]
    den = den_ref[0]
    anc = anc_ref[0]
    size_f = size_ref[0]

    eps = jnp.float32(1e-12)
    ssq = jnp.sum(cls * cls, axis=0, keepdims=True)
    inv_n = 1.0 / jnp.maximum(jnp.sqrt(ssq), eps)
    asq = jnp.sum(anc * anc, axis=1, keepdims=True)
    na = anc / jnp.maximum(jnp.sqrt(asq), eps)
    na_col = na.reshape(anc.shape[1], 1)

    sim = den * ((cls * inv_n) * na_col)

    bits = pltpu.bitcast(sim, jnp.uint32)
    ku = jnp.where(
        (bits >> 31) == jnp.uint32(0),
        bits | jnp.uint32(0x80000000),
        ~bits,
    )

    C = cls.shape[0]
    t = jnp.zeros((C, 1), jnp.uint32)
    for i in range(24):
        cand = t | jnp.uint32(1 << (31 - i))
        cnt = jnp.sum(jnp.where(ku >= cand, 1.0, 0.0), axis=1, keepdims=True)
        t = jnp.where(cnt >= size_f, cand, t)

    m_ge = ku >= t
    m_gt = ku > t
    sum_ge = jnp.sum(jnp.where(m_ge, cls, 0.0), axis=1, keepdims=True)
    sum_gt = jnp.sum(jnp.where(m_gt, cls, 0.0), axis=1, keepdims=True)
    cnt_ge = jnp.sum(jnp.where(m_ge, 1.0, 0.0), axis=1, keepdims=True)
    cnt_gt = jnp.sum(jnp.where(m_gt, 1.0, 0.0), axis=1, keepdims=True)
    eq_cnt = jnp.maximum(cnt_ge - cnt_gt, 1.0)
    need = size_f - cnt_gt
    out_ref[0] = sum_gt + (sum_ge - sum_gt) * (need / eq_cnt)


def _emb_body(size_ref, denom_ref, asum_ref, anc_ref, w1_ref, b1_ref,
              w2_ref, b2_ref, out_ref):
    size_f = size_ref[0, 0]
    denom = denom_ref[0, 0]
    asum = asum_ref[...]
    anc = anc_ref[...]
    C = asum.shape[1]
    dn = functools.partial(
        jax.lax.dot_general,
        dimension_numbers=(((1,), (1,)), ((), ())),
        preferred_element_type=jnp.float32,
        precision=jax.lax.Precision.HIGHEST,
    )
    emb1 = (dn(asum, w1_ref[...]) + size_f * b1_ref[...]) / denom
    w2a = w2_ref[:, :C]
    w2b = w2_ref[:, C:]
    out_ref[...] = dn(anc, w2a) + dn(emb1, w2b) + b2_ref[...]


def kernel(clsfea, denfea, anchor_cls, bs_mean, b, W_emb, b_emb, W_emb2,
           b_emb2, interpret=False):
    Bc, C, H, Wd = clsfea.shape
    K = H * Wd
    a2 = 384.0 * 576.0 / H / Wd
    prod = bs_mean[0, 0] * bs_mean[0, 1]
    size = jnp.floor_divide(prod.astype(jnp.float32),
                            jnp.float32(a2)).astype(jnp.int32)
    size = jnp.maximum(size, 3)
    size_f = size.astype(jnp.float32).reshape(1, 1)
    denom = (size.astype(jnp.float32)
             * (jnp.asarray(b, jnp.float32) / Bc)).reshape(1, 1)

    cls3 = clsfea.reshape(Bc, C, K)
    den3 = denfea.reshape(Bc, C, K)
    anc3 = anchor_cls.reshape(Bc, 1, C)

    B_SC = 4
    B_TC = Bc - B_SC

    keys = pl.pallas_call(
        _keys_body,
        grid=(B_SC,),
        in_specs=[
            pl.BlockSpec((1, C, K), lambda i: (i, 0, 0)),
            pl.BlockSpec((1, C, K), lambda i: (i, 0, 0)),
            pl.BlockSpec((1, 1, C), lambda i: (i, 0, 0)),
        ],
        out_specs=pl.BlockSpec((1, C, K), lambda i: (i, 0, 0)),
        out_shape=jax.ShapeDtypeStruct((B_SC, C, K), jnp.uint32),
        interpret=interpret,
    )(cls3[:B_SC], den3[:B_SC], anc3[:B_SC])

    nrows = B_SC * C
    R = nrows // (_NC * _NS)
    size_vec = jnp.full((_LANES,), size, jnp.int32)
    mesh = plsc.VectorSubcoreMesh(core_axis_name="c", subcore_axis_name="s")
    sc_select = pl.kernel(
        functools.partial(_sc_select_body, K, R),
        mesh=mesh,
        out_type=jax.ShapeDtypeStruct((nrows,), jnp.float32),
        scratch_types=[
            pltpu.VMEM((K,), jnp.uint32),
            pltpu.VMEM((K,), jnp.float32),
            pltpu.VMEM((R,), jnp.float32),
            pltpu.VMEM((_LANES,), jnp.int32),
        ],
    )
    asum_sc = sc_select(keys.reshape(nrows, K),
                        cls3[:B_SC].reshape(nrows, K), size_vec)

    asum_tc = pl.pallas_call(
        _tc_select_body,
        grid=(B_TC,),
        in_specs=[
            pl.BlockSpec((1, 1), lambda i: (0, 0)),
            pl.BlockSpec((1, C, K), lambda i: (i, 0, 0)),
            pl.BlockSpec((1, C, K), lambda i: (i, 0, 0)),
            pl.BlockSpec((1, 1, C), lambda i: (i, 0, 0)),
        ],
        out_specs=pl.BlockSpec((1, C, 1), lambda i: (i, 0, 0)),
        out_shape=jax.ShapeDtypeStruct((B_TC, C, 1), jnp.float32),
        interpret=interpret,
    )(size_f, cls3[B_SC:], den3[B_SC:], anc3[B_SC:])

    asum = jnp.concatenate(
        [asum_sc.reshape(B_SC, C), asum_tc.reshape(B_TC, C)], axis=0)

    out = pl.pallas_call(
        _emb_body,
        out_shape=jax.ShapeDtypeStruct((Bc, C), jnp.float32),
        interpret=interpret,
    )(size_f, denom, asum, anchor_cls.reshape(Bc, C),
      W_emb, b_emb.reshape(1, C), W_emb2, b_emb2.reshape(1, C))

    return out.reshape(Bc, C, 1, 1)
